# baseline (speedup 1.0000x reference)
import functools

import jax
import jax.numpy as jnp
from jax import lax
from jax.experimental import pallas as pl
from jax.experimental.pallas import tpu as pltpu
from jax.experimental.pallas import tpu_sc as plsc

NUM_NODES = 100000
EMBED_DIM = 128
BATCH = 16384

_info = plsc.get_sparse_core_info()
_NC = _info.num_cores
_NS = _info.num_subcores
_NW = _NC * _NS
_B_PER_W = BATCH // _NW

_NCHUNK = 4
_CH = _B_PER_W // _NCHUNK

_mesh = plsc.VectorSubcoreMesh(core_axis_name="c", subcore_axis_name="s")


@functools.partial(
    pl.kernel,
    mesh=_mesh,
    out_type=jax.ShapeDtypeStruct((BATCH, EMBED_DIM), jnp.float32),
    scratch_types=[
        pltpu.VMEM((_B_PER_W,), jnp.int32),
        pltpu.VMEM((_CH, EMBED_DIM), jnp.float32),
        pltpu.VMEM((_CH, EMBED_DIM), jnp.float32),
        pltpu.SemaphoreType.DMA,
        pltpu.SemaphoreType.DMA,
        pltpu.SemaphoreType.DMA,
        pltpu.SemaphoreType.DMA,
    ],
)
def _gather_kernel(table_hbm, idx_hbm, out_hbm, idx_v, buf0, buf1,
                   gsem0, gsem1, ssem0, ssem1):
    wid = lax.axis_index("s") * _NC + lax.axis_index("c")
    base = wid * _B_PER_W
    bufs = (buf0, buf1)
    gsems = (gsem0, gsem1)
    ssems = (ssem0, ssem1)

    pltpu.sync_copy(idx_hbm.at[pl.ds(base, _B_PER_W)], idx_v)

    gathers = [None] * _NCHUNK
    stores = [None] * _NCHUNK

    def issue_gather(i):
        return pltpu.async_copy(
            table_hbm.at[idx_v.at[pl.ds(i * _CH, _CH)]],
            bufs[i % 2], gsems[i % 2])

    gathers[0] = issue_gather(0)
    if _NCHUNK > 1:
        gathers[1] = issue_gather(1)

    for i in range(_NCHUNK):
        gathers[i].wait()
        stores[i] = pltpu.async_copy(
            bufs[i % 2],
            out_hbm.at[pl.ds(base + i * _CH, _CH)],
            ssems[i % 2])
        if i + 2 < _NCHUNK:
            stores[i].wait()
            gathers[i + 2] = issue_gather(i + 2)

    stores[-1].wait()
    if _NCHUNK > 1:
        stores[-2].wait()


def kernel(batch, embedding_weight):
    return _gather_kernel(embedding_weight, batch.astype(jnp.int32))


# speedup vs baseline: 1.0303x; 1.0303x over previous
import functools

import jax
import jax.numpy as jnp
from jax import lax
from jax.experimental import pallas as pl
from jax.experimental.pallas import tpu as pltpu
from jax.experimental.pallas import tpu_sc as plsc

NUM_NODES = 100000
EMBED_DIM = 128
BATCH = 16384

_info = plsc.get_sparse_core_info()
_NC = _info.num_cores
_NS = _info.num_subcores
_NW = _NC * _NS
_B_PER_W = BATCH // _NW

_NCHUNK = 2
_CH = _B_PER_W // _NCHUNK

_mesh = plsc.VectorSubcoreMesh(core_axis_name="c", subcore_axis_name="s")


@functools.partial(
    pl.kernel,
    mesh=_mesh,
    out_type=jax.ShapeDtypeStruct((BATCH, EMBED_DIM), jnp.float32),
    scratch_types=[
        pltpu.VMEM((_B_PER_W,), jnp.int32),
        pltpu.VMEM((_CH, EMBED_DIM), jnp.float32),
        pltpu.VMEM((_CH, EMBED_DIM), jnp.float32),
        pltpu.SemaphoreType.DMA,
        pltpu.SemaphoreType.DMA,
        pltpu.SemaphoreType.DMA,
        pltpu.SemaphoreType.DMA,
    ],
)
def _gather_kernel(table_hbm, idx_hbm, out_hbm, idx_v, buf0, buf1,
                   gsem0, gsem1, ssem0, ssem1):
    wid = lax.axis_index("s") * _NC + lax.axis_index("c")
    base = wid * _B_PER_W
    bufs = (buf0, buf1)
    gsems = (gsem0, gsem1)
    ssems = (ssem0, ssem1)

    pltpu.sync_copy(idx_hbm.at[pl.ds(base, _B_PER_W)], idx_v)

    gathers = [None] * _NCHUNK
    stores = [None] * _NCHUNK

    def issue_gather(i):
        return pltpu.async_copy(
            table_hbm.at[idx_v.at[pl.ds(i * _CH, _CH)]],
            bufs[i % 2], gsems[i % 2])

    gathers[0] = issue_gather(0)
    if _NCHUNK > 1:
        gathers[1] = issue_gather(1)

    for i in range(_NCHUNK):
        gathers[i].wait()
        stores[i] = pltpu.async_copy(
            bufs[i % 2],
            out_hbm.at[pl.ds(base + i * _CH, _CH)],
            ssems[i % 2])
        if i + 2 < _NCHUNK:
            stores[i].wait()
            gathers[i + 2] = issue_gather(i + 2)

    stores[-1].wait()
    if _NCHUNK > 1:
        stores[-2].wait()


def kernel(batch, embedding_weight):
    return _gather_kernel(embedding_weight, batch.astype(jnp.int32))
